# baseline (speedup 1.0000x reference)
import jax
import jax.numpy as jnp
from jax.experimental import pallas as pl

N_TOK = 16384
D_IN = 2048
D_H1 = 1024
D_H2 = 256
D_OUT = 2

TM = 1024


def _mlp_body(x_ref, w1_ref, b1_ref, w2_ref, b2_ref, w3_ref, b3_ref, o_ref):
    h = jnp.dot(x_ref[...], w1_ref[...], preferred_element_type=jnp.float32)
    h = jnp.maximum(h + b1_ref[...][None, :], 0.0)
    h = jnp.dot(h, w2_ref[...], preferred_element_type=jnp.float32)
    h = jnp.maximum(h + b2_ref[...][None, :], 0.0)
    o = jnp.dot(h, w3_ref[...], preferred_element_type=jnp.float32)
    o_ref[...] = o + b3_ref[...][None, :]


def kernel(input, W1, b1, W2, b2, W3, b3):
    n = input.shape[0]
    grid = (n // TM,)

    out = pl.pallas_call(
        _mlp_body,
        grid=grid,
        in_specs=[
            pl.BlockSpec((TM, D_IN), lambda i: (i, 0)),
            pl.BlockSpec((D_IN, D_H1), lambda i: (0, 0)),
            pl.BlockSpec((D_H1,), lambda i: (0,)),
            pl.BlockSpec((D_H1, D_H2), lambda i: (0, 0)),
            pl.BlockSpec((D_H2,), lambda i: (0,)),
            pl.BlockSpec((D_H2, D_OUT), lambda i: (0, 0)),
            pl.BlockSpec((D_OUT,), lambda i: (0,)),
        ],
        out_specs=pl.BlockSpec((TM, D_OUT), lambda i: (i, 0)),
        out_shape=jax.ShapeDtypeStruct((n, D_OUT), jnp.float32),
    )(input, W1, b1, W2, b2, W3, b3)
    return out


# speedup vs baseline: 1.0638x; 1.0638x over previous
import jax
import jax.numpy as jnp
from jax.experimental import pallas as pl

N_TOK = 16384
D_IN = 2048
D_H1 = 1024
D_H2 = 256
D_OUT = 2

TM = 1024


def _mlp_body(x_ref, w1_ref, b1_ref, w2_ref, b2_ref, w3_ref, b3_ref, o_ref):
    i = pl.program_id(0)
    h = jnp.dot(x_ref[...], w1_ref[...], preferred_element_type=jnp.float32)
    h = jnp.maximum(h + b1_ref[...][None, :], 0.0)
    h = jnp.dot(h, w2_ref[...], preferred_element_type=jnp.float32)
    h = jnp.maximum(h + b2_ref[...][None, :], 0.0)
    o = jnp.dot(h, w3_ref[...], preferred_element_type=jnp.float32)
    o = o + b3_ref[...][None, :]
    o_ref[:, pl.ds(i * TM, TM)] = o.T


def kernel(input, W1, b1, W2, b2, W3, b3):
    n = input.shape[0]
    grid = (n // TM,)

    out_t = pl.pallas_call(
        _mlp_body,
        grid=grid,
        in_specs=[
            pl.BlockSpec((TM, D_IN), lambda i: (i, 0)),
            pl.BlockSpec((D_IN, D_H1), lambda i: (0, 0)),
            pl.BlockSpec((D_H1,), lambda i: (0,)),
            pl.BlockSpec((D_H1, D_H2), lambda i: (0, 0)),
            pl.BlockSpec((D_H2,), lambda i: (0,)),
            pl.BlockSpec((D_H2, D_OUT), lambda i: (0, 0)),
            pl.BlockSpec((D_OUT,), lambda i: (0,)),
        ],
        out_specs=pl.BlockSpec((D_OUT, n), lambda i: (0, 0)),
        out_shape=jax.ShapeDtypeStruct((D_OUT, n), jnp.float32),
    )(input, W1, b1, W2, b2, W3, b3)
    return out_t.T


# speedup vs baseline: 1.0817x; 1.0168x over previous
import jax
import jax.numpy as jnp
from jax.experimental import pallas as pl

N_TOK = 16384
D_IN = 2048
D_H1 = 1024
D_H2 = 256
D_OUT = 2

TM = 1024


def _mlp_body(x_ref, w1_ref, b1_ref, w2_ref, b2_ref, w3t_ref, b3_ref, o_ref):
    i = pl.program_id(0)
    h = jnp.dot(x_ref[...], w1_ref[...], preferred_element_type=jnp.float32)
    h = jnp.maximum(h + b1_ref[...][None, :], 0.0)
    h = jnp.dot(h, w2_ref[...], preferred_element_type=jnp.float32)
    h = jnp.maximum(h + b2_ref[...][None, :], 0.0)
    o = jax.lax.dot_general(
        h, w3t_ref[...], (((1,), (1,)), ((), ())), preferred_element_type=jnp.float32
    )
    o = o + b3_ref[...][None, :]
    o_ref[:, pl.ds(i * TM, TM)] = o.T


def kernel(input, W1, b1, W2, b2, W3, b3):
    n = input.shape[0]
    grid = (n // TM,)

    out_t = pl.pallas_call(
        _mlp_body,
        grid=grid,
        in_specs=[
            pl.BlockSpec((TM, D_IN), lambda i: (i, 0)),
            pl.BlockSpec((D_IN, D_H1), lambda i: (0, 0)),
            pl.BlockSpec((D_H1,), lambda i: (0,)),
            pl.BlockSpec((D_H1, D_H2), lambda i: (0, 0)),
            pl.BlockSpec((D_H2,), lambda i: (0,)),
            pl.BlockSpec((D_OUT, D_H2), lambda i: (0, 0)),
            pl.BlockSpec((D_OUT,), lambda i: (0,)),
        ],
        out_specs=pl.BlockSpec((D_OUT, n), lambda i: (0, 0)),
        out_shape=jax.ShapeDtypeStruct((D_OUT, n), jnp.float32),
    )(input, W1, b1, W2, b2, W3.T, b3)
    return out_t.T


# speedup vs baseline: 1.0818x; 1.0001x over previous
import jax
import jax.numpy as jnp
from jax.experimental import pallas as pl

N_TOK = 16384
D_IN = 2048
D_H1 = 1024
D_H2 = 256
D_OUT = 2

TM = 2048


def _mlp_body(x_ref, w1_ref, b1_ref, w2_ref, b2_ref, w3t_ref, b3_ref, o_ref):
    i = pl.program_id(0)
    h = jnp.dot(x_ref[...], w1_ref[...], preferred_element_type=jnp.float32)
    h = jnp.maximum(h + b1_ref[...][None, :], 0.0)
    h = jnp.dot(h, w2_ref[...], preferred_element_type=jnp.float32)
    h = jnp.maximum(h + b2_ref[...][None, :], 0.0)
    o = jax.lax.dot_general(
        h, w3t_ref[...], (((1,), (1,)), ((), ())), preferred_element_type=jnp.float32
    )
    o = o + b3_ref[...][None, :]
    o_ref[:, pl.ds(i * TM, TM)] = o.T


def kernel(input, W1, b1, W2, b2, W3, b3):
    n = input.shape[0]
    grid = (n // TM,)

    out_t = pl.pallas_call(
        _mlp_body,
        grid=grid,
        in_specs=[
            pl.BlockSpec((TM, D_IN), lambda i: (i, 0)),
            pl.BlockSpec((D_IN, D_H1), lambda i: (0, 0)),
            pl.BlockSpec((D_H1,), lambda i: (0,)),
            pl.BlockSpec((D_H1, D_H2), lambda i: (0, 0)),
            pl.BlockSpec((D_H2,), lambda i: (0,)),
            pl.BlockSpec((D_OUT, D_H2), lambda i: (0, 0)),
            pl.BlockSpec((D_OUT,), lambda i: (0,)),
        ],
        out_specs=pl.BlockSpec((D_OUT, n), lambda i: (0, 0)),
        out_shape=jax.ShapeDtypeStruct((D_OUT, n), jnp.float32),
    )(input, W1, b1, W2, b2, W3.T, b3)
    return out_t.T
